# baseline (device time: 101961 ns/iter reference)
import jax
import jax.numpy as jnp
from jax import lax
from jax.experimental import pallas as pl
from jax.experimental.pallas import tpu as pltpu

P = 16
SUB = 4
NSLOT = 4
WIRE_DTYPE = jnp.bfloat16

RING = [0, 3, 7, 11, 15, 14, 10, 6, 2, 1, 5, 9, 13, 12, 8, 4]
POS = [0] * P
for _r, _l in enumerate(RING):
    POS[_l] = _r
NEXT = [RING[(POS[l] + 1) % P] for l in range(P)]
PREV = [RING[(POS[l] - 1) % P] for l in range(P)]


def kernel(x, w_mat, scale_x, scale_w):
    m_glob, k_per = x.shape
    _, n = w_mat.shape
    m_chunk = m_glob // P
    nh = n // 2
    ns = nh // SUB

    my = lax.axis_index("i")
    pos = jnp.asarray(POS, jnp.int32)[my].reshape(1)
    nxt = jnp.asarray(NEXT, jnp.int32)[my].reshape(1)
    prv = jnp.asarray(PREV, jnp.int32)[my].reshape(1)
    ring = jnp.asarray(RING, jnp.int32)

    def body(x_ref, w_ref, sx_ref, sw_ref, pos_ref, nxt_ref, prv_ref,
             ring_ref, out_ref, s_cw, s_ccw, comm_cw, comm_ccw,
             send_sems, recv_sems, credit_sems):
        pos = pos_ref[0]
        nxt = nxt_ref[0]
        prv = prv_ref[0]

        def chunk_gemm(c, col0, col1):
            xs = x_ref[pl.ds(c * m_chunk, m_chunk), :]
            return lax.dot_general(
                xs, w_ref[:, col0:col1], (((1,), (0,)), ((), ())),
                preferred_element_type=jnp.float32)

        def mk(d, h, s, target):
            buf = s_cw if d == 0 else s_ccw
            com = comm_cw if d == 0 else comm_ccw
            return pltpu.make_async_remote_copy(
                src_ref=buf.at[h % 2, s],
                dst_ref=com.at[h % NSLOT, s],
                send_sem=send_sems.at[d, h % 2, s],
                recv_sem=recv_sems.at[d, h % NSLOT, s],
                device_id=(target,),
                device_id_type=pl.DeviceIdType.MESH,
            )

        barrier = pltpu.get_barrier_semaphore()
        for nbr in (prv, nxt):
            pl.semaphore_signal(barrier, inc=1, device_id=(nbr,),
                                device_id_type=pl.DeviceIdType.MESH)
        pl.semaphore_wait(barrier, 2)

        for h in range(P - 1):
            pslot = (h - 1) % NSLOT
            c_cw = ring_ref[lax.rem(pos + 2 * P - h - 1, P)]
            c_ccw = ring_ref[lax.rem(pos + h + 1, P)]
            l_cw = chunk_gemm(c_cw, 0, nh)
            l_ccw = chunk_gemm(c_ccw, nh, n)
            if h >= NSLOT:
                pl.semaphore_wait(credit_sems.at[0, h % NSLOT], 1)
                pl.semaphore_wait(credit_sems.at[1, h % NSLOT], 1)
            for s in range(SUB):
                if h > 0:
                    mk(0, h - 1, s, nxt).wait_recv()
                    mk(1, h - 1, s, prv).wait_recv()
                    v_cw = comm_cw[pslot, s].astype(jnp.float32) \
                        + l_cw[:, s * ns:(s + 1) * ns]
                    v_ccw = comm_ccw[pslot, s].astype(jnp.float32) \
                        + l_ccw[:, s * ns:(s + 1) * ns]
                else:
                    v_cw = l_cw[:, s * ns:(s + 1) * ns]
                    v_ccw = l_ccw[:, s * ns:(s + 1) * ns]
                if h >= 2:
                    mk(0, h - 2, s, nxt).wait_send()
                    mk(1, h - 2, s, prv).wait_send()
                s_cw[h % 2, s] = v_cw.astype(WIRE_DTYPE)
                s_ccw[h % 2, s] = v_ccw.astype(WIRE_DTYPE)
                mk(0, h, s, nxt).start()
                mk(1, h, s, prv).start()
            if 1 <= h <= P - 2 - (NSLOT - 1):
                pl.semaphore_signal(credit_sems.at[0, pslot], inc=1,
                                    device_id=(prv,),
                                    device_id_type=pl.DeviceIdType.MESH)
                pl.semaphore_signal(credit_sems.at[1, pslot], inc=1,
                                    device_id=(nxt,),
                                    device_id_type=pl.DeviceIdType.MESH)

        scale = sx_ref[0] * sw_ref[0]
        c_fin = ring_ref[pos]
        f_cw = chunk_gemm(c_fin, 0, nh)
        f_ccw = chunk_gemm(c_fin, nh, n)
        fslot = (P - 2) % NSLOT
        for s in range(SUB):
            mk(0, P - 2, s, nxt).wait_recv()
            mk(1, P - 2, s, prv).wait_recv()
            v_cw = comm_cw[fslot, s].astype(jnp.float32) \
                + f_cw[:, s * ns:(s + 1) * ns]
            v_ccw = comm_ccw[fslot, s].astype(jnp.float32) \
                + f_ccw[:, s * ns:(s + 1) * ns]
            out_ref[:, s * ns:(s + 1) * ns] = jnp.maximum(v_cw * scale, 0.0)
            out_ref[:, nh + s * ns:nh + (s + 1) * ns] = \
                jnp.maximum(v_ccw * scale, 0.0)
        for hh in (P - 3, P - 2):
            for s in range(SUB):
                mk(0, hh, s, nxt).wait_send()
                mk(1, hh, s, prv).wait_send()

    return pl.pallas_call(
        body,
        out_shape=jax.ShapeDtypeStruct((m_chunk, n), jnp.float32),
        in_specs=[
            pl.BlockSpec(memory_space=pltpu.VMEM),
            pl.BlockSpec(memory_space=pltpu.VMEM),
            pl.BlockSpec(memory_space=pltpu.SMEM),
            pl.BlockSpec(memory_space=pltpu.SMEM),
            pl.BlockSpec(memory_space=pltpu.SMEM),
            pl.BlockSpec(memory_space=pltpu.SMEM),
            pl.BlockSpec(memory_space=pltpu.SMEM),
            pl.BlockSpec(memory_space=pltpu.SMEM),
        ],
        out_specs=pl.BlockSpec(memory_space=pltpu.VMEM),
        scratch_shapes=[
            pltpu.VMEM((2, SUB, m_chunk, ns), WIRE_DTYPE),
            pltpu.VMEM((2, SUB, m_chunk, ns), WIRE_DTYPE),
            pltpu.VMEM((NSLOT, SUB, m_chunk, ns), WIRE_DTYPE),
            pltpu.VMEM((NSLOT, SUB, m_chunk, ns), WIRE_DTYPE),
            pltpu.SemaphoreType.DMA((2, 2, SUB)),
            pltpu.SemaphoreType.DMA((2, NSLOT, SUB)),
            pltpu.SemaphoreType.REGULAR((2, NSLOT)),
        ],
        compiler_params=pltpu.CompilerParams(collective_id=0),
    )(x, w_mat, scale_x, scale_w, pos, nxt, prv, ring)
